# Initial kernel scaffold; baseline (speedup 1.0000x reference)
import jax
import jax.numpy as jnp
from jax.experimental import pallas as pl


def kernel(x, y, x_feat, y_feat, parent, rank, W1, b1, W2, b2):
    raise NotImplementedError("write your pallas kernel here")


# speedup vs baseline: 1.9139x; 1.9139x over previous
import jax
import jax.numpy as jnp
from jax.experimental import pallas as pl

_SIM_THRESHOLD = 0.7


def _attn_kernel(xf_ref, yf_ref, W1_ref, b1_ref, W2_ref, b2_ref, out_ref):
    xf = xf_ref[...]
    yf = yf_ref[...]
    eps = 1e-8
    nx = jnp.maximum(jnp.sqrt(jnp.sum(xf * xf)), eps)
    ny = jnp.maximum(jnp.sqrt(jnp.sum(yf * yf)), eps)
    sim = jnp.sum(xf * yf) / (nx * ny)

    h = (jnp.dot(xf, W1_ref[0:128, :], preferred_element_type=jnp.float32)
         + jnp.dot(yf, W1_ref[128:256, :], preferred_element_type=jnp.float32)
         + b1_ref[...])
    h = jnp.maximum(h, 0.0)
    a = jnp.dot(h, W2_ref[...], preferred_element_type=jnp.float32) + b2_ref[...]
    attn = jax.nn.sigmoid(a)
    out_ref[...] = jnp.where(sim < _SIM_THRESHOLD, jnp.zeros_like(attn), attn)


def kernel(x, y, x_feat, y_feat, parent, rank, W1, b1, W2, b2):
    out = pl.pallas_call(
        _attn_kernel,
        out_shape=jax.ShapeDtypeStruct((1, 1), jnp.float32),
    )(
        x_feat.reshape(1, 128),
        y_feat.reshape(1, 128),
        W1,
        b1.reshape(1, 128),
        W2,
        b2.reshape(1, 1),
    )
    return out.reshape(1)
